# baseline (speedup 1.0000x reference)
import jax
import jax.numpy as jnp
from jax import lax
from jax.experimental import pallas as pl
from jax.experimental.pallas import tpu as pltpu
from jax.experimental.pallas import tpu_sc as plsc

S = 256
N_TRIPLES = 4096
TRIPLE_H = 512
LLM_H = 2048
TOTAL_TOK = 8192
GATE_BOOST = 1.0

NC, NS = 2, 16
NW = NC * NS
TOK_W = TOTAL_TOK // NW
K = 16
NCH = TOK_W // K
SROW_W = S // NW
TBK = 512
NBLK = TOTAL_TOK // TBK

f32 = jnp.float32
i32 = jnp.int32


def _stage1_body(te, gw_row, k_row, sel_row, cu_lo_row, cu_hi_row,
                 W1, b1, W2, b2, ln_g, ln_b, Wo1, bo1, Wo2, bo2,
                 enh_o, spos_o, src_o, dest_o, g_o):
    isub = lax.broadcasted_iota(i32, (S, S), 0).astype(f32)
    ilan = lax.broadcasted_iota(i32, (S, S), 1).astype(f32)
    ident = (isub == ilan).astype(f32)

    def to_col(row):
        return lax.dot_general(ident, row, (((1,), (1,)), ((), ())),
                               preferred_element_type=f32)

    kr = k_row[...]
    kc = to_col(kr)
    cu_lo_r, cu_hi_r = cu_lo_row[...], cu_hi_row[...]
    lens_row = cu_hi_r - cu_lo_r
    lens_col = to_col(cu_hi_r) - to_col(cu_lo_r)

    vsub = lax.broadcasted_iota(i32, (N_TRIPLES, S), 0)
    onehot_selT = (sel_row[...].astype(i32) == vsub).astype(f32)
    selected = lax.dot_general(onehot_selT, te[...], (((0,), (0,)), ((), ())),
                               preferred_element_type=f32)
    g_raw_row = lax.dot_general(gw_row[...], onehot_selT,
                                (((1,), (0,)), ((), ())),
                                preferred_element_type=f32)

    h = jnp.maximum(jnp.dot(selected, W1[...], preferred_element_type=f32)
                    + b1[...], 0.0)
    h = jnp.dot(h, W2[...], preferred_element_type=f32) + b2[...]
    mu = jnp.mean(h, axis=-1, keepdims=True)
    var = jnp.mean((h - mu) ** 2, axis=-1, keepdims=True)
    sp = (h - mu) * lax.rsqrt(var + 1e-5) * ln_g[...] + ln_b[...]

    Mc = (kr < kc) | ((kr == kc) & (ilan < isub))
    rank_col = jnp.sum(Mc.astype(f32), axis=1, keepdims=True)
    Mr = (kc < kr) | ((kc == kr) & (isub < ilan))
    rank_row = jnp.sum(Mr.astype(f32), axis=0, keepdims=True)

    PT = (rank_col == ilan).astype(f32)
    Ppi = (rank_row == isub).astype(f32)

    scnt_col = jnp.sum(Ppi * kr, axis=1, keepdims=True)
    graw_s_col = jnp.sum(Ppi * g_raw_row, axis=1, keepdims=True)
    a_col = jnp.sum(Ppi * cu_lo_r, axis=1, keepdims=True)
    slens_row = jnp.sum(PT * lens_col, axis=0, keepdims=True)
    slens_col = jnp.sum(Ppi * lens_row, axis=1, keepdims=True)

    sorted_struct = lax.dot_general(PT, sp, (((0,), (0,)), ((), ())),
                                    preferred_element_type=f32)

    ncu_row = jnp.sum((isub <= ilan).astype(f32) * slens_col,
                      axis=0, keepdims=True)
    ncu_col = jnp.sum((ilan <= isub).astype(f32) * slens_row,
                      axis=1, keepdims=True)
    start_row = ncu_row - slens_row
    start_col = ncu_col - slens_col
    spos_row = start_row + lax.broadcasted_iota(i32, (1, S), 1).astype(f32)
    spos_o[...] = spos_row.astype(i32)

    maxc = jnp.maximum(jnp.max(kr), 1.0)
    so_col = scnt_col / maxc
    ob = jnp.maximum(so_col * Wo1[...] + bo1[...], 0.0)
    ob = jnp.dot(ob, Wo2[...], preferred_element_type=f32) + bo2[...]
    lowmask_col = scnt_col < 2.0
    ob = jnp.where(lowmask_col, 0.0, ob)
    enh_o[...] = sorted_struct + ob

    g_sel_col = jnp.where(lowmask_col, 1.0, 1.0 + GATE_BOOST * graw_s_col)

    isub_col = lax.broadcasted_iota(i32, (S, 1), 0).astype(f32)
    for blk in range(NBLK):
        t_row = (lax.broadcasted_iota(i32, (1, TBK), 1).astype(f32)
                 + float(blk * TBK))
        ind = (t_row >= ncu_col).astype(f32)
        seg_row = jnp.sum(ind, axis=0, keepdims=True)
        onehot = (seg_row == isub_col).astype(f32)
        start_at = jnp.sum(onehot * start_col, axis=0, keepdims=True)
        a_at = jnp.sum(onehot * a_col, axis=0, keepdims=True)
        g_at = jnp.sum(onehot * g_sel_col, axis=0, keepdims=True)
        row = pl.ds(blk, 1)
        src_o[row, :] = (a_at + t_row - start_at).astype(i32)
        dest_o[row, :] = (t_row + seg_row + 1.0).astype(i32)
        g_o[row, :] = g_at


def _sc_body(embed, tok_ids, src2, dest2, g2, enh, spos2, out,
             tok_v, src_v, g_v, dest_v, ids2_v, dest16_v, rows_v,
             struct_v, spos_full_v, spos16_v, gsem, ssem, hsem):
    wid = lax.axis_index("s") * NC + lax.axis_index("c")
    row = wid // 2
    off = (wid % 2) * TOK_W

    pltpu.sync_copy(tok_ids, tok_v)
    pltpu.sync_copy(src2.at[pl.ds(row, 1), pl.ds(off, TOK_W)], src_v)
    pltpu.sync_copy(g2.at[pl.ds(row, 1), pl.ds(off, TOK_W)], g_v)
    pltpu.sync_copy(dest2.at[pl.ds(row, 1), pl.ds(off, TOK_W)], dest_v)
    pltpu.sync_copy(spos2, spos_full_v)

    for c in range(NCH):
        idx = src_v[0, pl.ds(c * 16, 16)]
        ids2_v[c] = plsc.load_gather(tok_v, [idx])
        dest16_v[c] = dest_v[0, pl.ds(c * 16, 16)]

    zeros16 = jnp.zeros((16,), i32)
    gathers = [None, None]
    scatters = [None, None]
    gathers[0] = pltpu.async_copy(embed.at[ids2_v.at[0]], rows_v.at[0], gsem)
    for c in range(NCH):
        b, nb = c % 2, (c + 1) % 2
        if c + 1 < NCH:
            if scatters[nb] is not None:
                scatters[nb].wait()
                scatters[nb] = None
            gathers[nb] = pltpu.async_copy(
                embed.at[ids2_v.at[c + 1]], rows_v.at[nb], gsem)
        gathers[b].wait()
        gates = [plsc.load_gather(g_v, [zeros16,
                                        jnp.full((16,), c * K + r, i32)])
                 for r in range(K)]

        def mul(j, _, b=b, gates=gates):
            for r in range(K):
                rows_v[b, r, pl.ds(j * 16, 16)] = (
                    rows_v[b, r, pl.ds(j * 16, 16)] * gates[r])
            return 0

        lax.fori_loop(0, LLM_H // 16, mul, 0)
        scatters[b] = pltpu.async_copy(rows_v.at[b], out.at[dest16_v.at[c]],
                                       ssem)

    @pl.when(wid < 16)
    def _():
        spos16_v[0] = spos_full_v[0, pl.ds(wid * 16, 16)]
        pltpu.async_copy(enh.at[pl.ds(wid * 16, 16)], struct_v, hsem).wait()
        pltpu.async_copy(struct_v, out.at[spos16_v.at[0]], hsem).wait()

    for s in scatters:
        if s is not None:
            s.wait()


def _build_sc(interpret=False):
    mesh = plsc.VectorSubcoreMesh(core_axis_name="c", subcore_axis_name="s",
                                  num_cores=NC, num_subcores=NS)
    return pl.kernel(
        _sc_body,
        out_type=jax.ShapeDtypeStruct((S + TOTAL_TOK, LLM_H), f32),
        mesh=mesh,
        scratch_types=[
            pltpu.VMEM((TOTAL_TOK,), i32),
            pltpu.VMEM((1, TOK_W), i32),
            pltpu.VMEM((1, TOK_W), f32),
            pltpu.VMEM((1, TOK_W), i32),
            pltpu.VMEM((NCH, K), i32),
            pltpu.VMEM((NCH, K), i32),
            pltpu.VMEM((2, K, LLM_H), f32),
            pltpu.VMEM((16, LLM_H), f32),
            pltpu.VMEM((1, S), i32),
            pltpu.VMEM((1, 16), i32),
            pltpu.SemaphoreType.DMA,
            pltpu.SemaphoreType.DMA,
            pltpu.SemaphoreType.DMA,
        ],
        compiler_params=pltpu.CompilerParams(needs_layout_passes=False),
        interpret=interpret,
    )


def kernel(triple_embeds, overlap_cnt, sel_indices, gate_weights, token_ids,
           cu_seqlens, W1, b1, W2, b2, ln_g, ln_b, Wo1, bo1, Wo2, bo2,
           embed_table, *, interpret=False):
    ocf = overlap_cnt.astype(f32)
    cu_f = cu_seqlens.astype(f32)

    sds = jax.ShapeDtypeStruct
    enh, spos_row, src2, dest2, g2 = pl.pallas_call(
        _stage1_body,
        out_shape=(sds((S, LLM_H), f32), sds((1, S), i32),
                   sds((NBLK, TBK), i32), sds((NBLK, TBK), i32),
                   sds((NBLK, TBK), f32)),
        interpret=interpret,
    )(triple_embeds, gate_weights[None, :], ocf[None, :],
      sel_indices[None, :].astype(f32), cu_f[None, :S], cu_f[None, 1:],
      W1, b1[None, :], W2, b2[None, :], ln_g[None, :], ln_b[None, :],
      Wo1, bo1[None, :], Wo2, bo2[None, :])

    out = _build_sc(interpret=interpret)(
        embed_table, token_ids, src2, dest2, g2, enh, spos_row)
    return out[None, :, :]


# speedup vs baseline: 1.0651x; 1.0651x over previous
import jax
import jax.numpy as jnp
from jax import lax
from jax.experimental import pallas as pl
from jax.experimental.pallas import tpu as pltpu
from jax.experimental.pallas import tpu_sc as plsc

S = 256
N_TRIPLES = 4096
TRIPLE_H = 512
LLM_H = 2048
TOTAL_TOK = 8192
GATE_BOOST = 1.0

NC, NS = 2, 16
NW = NC * NS
TOK_W = TOTAL_TOK // NW
K = 16
NCH = TOK_W // K
SROW_W = S // NW
TBK = 512
NBLK = TOTAL_TOK // TBK

f32 = jnp.float32
i32 = jnp.int32


def _stage1_body(te, gw_row, k_row, sel_row, cu_row,
                 W1, b1, W2, b2, ln_g, ln_b, Wo1, bo1, Wo2, bo2,
                 enh_o, spos_o, src_o, dest_o, g_o):
    isub = lax.broadcasted_iota(i32, (S, S), 0).astype(f32)
    ilan = lax.broadcasted_iota(i32, (S, S), 1).astype(f32)
    ident = (isub == ilan).astype(f32)

    def to_col(row):
        return lax.dot_general(ident, row, (((1,), (1,)), ((), ())),
                               preferred_element_type=f32)

    kr = k_row[...].astype(f32)
    kc = to_col(kr)
    cu_f = cu_row[...].astype(f32)
    cu_lo_r, cu_hi_r = cu_f[:, :S], cu_f[:, 1:]
    lens_row = cu_hi_r - cu_lo_r
    lens_col = to_col(cu_hi_r) - to_col(cu_lo_r)

    vsub = lax.broadcasted_iota(i32, (N_TRIPLES, S), 0)
    onehot_selT = (sel_row[...] == vsub).astype(f32)
    selected = lax.dot_general(onehot_selT, te[...], (((0,), (0,)), ((), ())),
                               preferred_element_type=f32)
    g_raw_row = lax.dot_general(gw_row[...], onehot_selT,
                                (((1,), (0,)), ((), ())),
                                preferred_element_type=f32)

    h = jnp.maximum(jnp.dot(selected, W1[...], preferred_element_type=f32)
                    + b1[...], 0.0)
    h = jnp.dot(h, W2[...], preferred_element_type=f32) + b2[...]
    mu = jnp.mean(h, axis=-1, keepdims=True)
    var = jnp.mean((h - mu) ** 2, axis=-1, keepdims=True)
    sp = (h - mu) * lax.rsqrt(var + 1e-5) * ln_g[...] + ln_b[...]

    Mc = (kr < kc) | ((kr == kc) & (ilan < isub))
    rank_col = jnp.sum(Mc.astype(f32), axis=1, keepdims=True)
    Mr = (kc < kr) | ((kc == kr) & (isub < ilan))
    rank_row = jnp.sum(Mr.astype(f32), axis=0, keepdims=True)

    PT = (rank_col == ilan).astype(f32)
    Ppi = (rank_row == isub).astype(f32)

    scnt_col = jnp.sum(Ppi * kr, axis=1, keepdims=True)
    graw_s_col = jnp.sum(Ppi * g_raw_row, axis=1, keepdims=True)
    a_col = jnp.sum(Ppi * cu_lo_r, axis=1, keepdims=True)
    slens_row = jnp.sum(PT * lens_col, axis=0, keepdims=True)
    slens_col = jnp.sum(Ppi * lens_row, axis=1, keepdims=True)

    sorted_struct = lax.dot_general(PT, sp, (((0,), (0,)), ((), ())),
                                    preferred_element_type=f32)

    ncu_row = jnp.sum((isub <= ilan).astype(f32) * slens_col,
                      axis=0, keepdims=True)
    ncu_col = jnp.sum((ilan <= isub).astype(f32) * slens_row,
                      axis=1, keepdims=True)
    start_row = ncu_row - slens_row
    start_col = ncu_col - slens_col
    spos_row = start_row + lax.broadcasted_iota(i32, (1, S), 1).astype(f32)
    spos_o[...] = spos_row.astype(i32)

    maxc = jnp.maximum(jnp.max(kr), 1.0)
    so_col = scnt_col / maxc
    ob = jnp.maximum(so_col * Wo1[...] + bo1[...], 0.0)
    ob = jnp.dot(ob, Wo2[...], preferred_element_type=f32) + bo2[...]
    lowmask_col = scnt_col < 2.0
    ob = jnp.where(lowmask_col, 0.0, ob)
    enh_o[...] = sorted_struct + ob

    g_sel_col = jnp.where(lowmask_col, 1.0, 1.0 + GATE_BOOST * graw_s_col)

    isub_col = lax.broadcasted_iota(i32, (S, 1), 0).astype(f32)
    for blk in range(NBLK):
        t_row = (lax.broadcasted_iota(i32, (1, TBK), 1).astype(f32)
                 + float(blk * TBK))
        ind = (t_row >= ncu_col).astype(f32)
        seg_row = jnp.sum(ind, axis=0, keepdims=True)
        onehot = (seg_row == isub_col).astype(f32)
        start_at = jnp.sum(onehot * start_col, axis=0, keepdims=True)
        a_at = jnp.sum(onehot * a_col, axis=0, keepdims=True)
        g_at = jnp.sum(onehot * g_sel_col, axis=0, keepdims=True)
        row = pl.ds(blk, 1)
        src_o[row, :] = (a_at + t_row - start_at).astype(i32)
        dest_o[row, :] = (t_row + seg_row + 1.0).astype(i32)
        g_o[row, :] = g_at


def _sc_body(embed, tok_ids, src2, dest2, g2, enh, spos2, out,
             tok_v, src_v, g_v, dest_v, ids2_v, dest16_v, rows_v,
             spos_full_v, spos16_v, gsem, ssem, hsem):
    wid = lax.axis_index("s") * NC + lax.axis_index("c")
    row = wid // 2
    off = (wid % 2) * TOK_W

    pltpu.sync_copy(tok_ids, tok_v)
    pltpu.sync_copy(src2.at[pl.ds(row, 1), pl.ds(off, TOK_W)], src_v)
    pltpu.sync_copy(g2.at[pl.ds(row, 1), pl.ds(off, TOK_W)], g_v)
    pltpu.sync_copy(dest2.at[pl.ds(row, 1), pl.ds(off, TOK_W)], dest_v)
    pltpu.sync_copy(spos2, spos_full_v)

    for c in range(NCH):
        idx = src_v[0, pl.ds(c * 16, 16)]
        ids2_v[c] = plsc.load_gather(tok_v, [idx])
        dest16_v[c] = dest_v[0, pl.ds(c * 16, 16)]

    NBUF = 3
    zeros16 = jnp.zeros((16,), i32)
    gathers = [None] * NBUF
    scatters = [None] * NBUF

    def issue_gather(c):
        b = c % NBUF
        if scatters[b] is not None:
            scatters[b].wait()
            scatters[b] = None
        gathers[b] = pltpu.async_copy(embed.at[ids2_v.at[c]], rows_v.at[b],
                                      gsem)

    issue_gather(0)
    issue_gather(1)
    for c in range(NCH):
        b = c % NBUF
        if c + 2 < NCH:
            issue_gather(c + 2)
        gathers[b].wait()
        gates = [plsc.load_gather(g_v, [zeros16,
                                        jnp.full((16,), c * K + r, i32)])
                 for r in range(K)]

        def mul(j, _, b=b, gates=gates):
            for r in range(K):
                rows_v[b, r, pl.ds(j * 16, 16)] = (
                    rows_v[b, r, pl.ds(j * 16, 16)] * gates[r])
            return 0

        lax.fori_loop(0, LLM_H // 16, mul, 0)
        scatters[b] = pltpu.async_copy(rows_v.at[b], out.at[dest16_v.at[c]],
                                       ssem)

    for s in scatters:
        if s is not None:
            s.wait()

    @pl.when(wid < 16)
    def _():
        spos16_v[0] = spos_full_v[0, pl.ds(wid * 16, 16)]
        pltpu.async_copy(enh.at[pl.ds(wid * 16, 16)], rows_v.at[0],
                         hsem).wait()
        pltpu.async_copy(rows_v.at[0], out.at[spos16_v.at[0]], hsem).wait()


def _build_sc(interpret=False):
    mesh = plsc.VectorSubcoreMesh(core_axis_name="c", subcore_axis_name="s",
                                  num_cores=NC, num_subcores=NS)
    return pl.kernel(
        _sc_body,
        out_type=jax.ShapeDtypeStruct((S + TOTAL_TOK, LLM_H), f32),
        mesh=mesh,
        scratch_types=[
            pltpu.VMEM((TOTAL_TOK,), i32),
            pltpu.VMEM((1, TOK_W), i32),
            pltpu.VMEM((1, TOK_W), f32),
            pltpu.VMEM((1, TOK_W), i32),
            pltpu.VMEM((NCH, K), i32),
            pltpu.VMEM((NCH, K), i32),
            pltpu.VMEM((3, K, LLM_H), f32),
            pltpu.VMEM((1, S), i32),
            pltpu.VMEM((1, 16), i32),
            pltpu.SemaphoreType.DMA,
            pltpu.SemaphoreType.DMA,
            pltpu.SemaphoreType.DMA,
        ],
        compiler_params=pltpu.CompilerParams(needs_layout_passes=False),
        interpret=interpret,
    )


def kernel(triple_embeds, overlap_cnt, sel_indices, gate_weights, token_ids,
           cu_seqlens, W1, b1, W2, b2, ln_g, ln_b, Wo1, bo1, Wo2, bo2,
           embed_table, *, interpret=False):
    sds = jax.ShapeDtypeStruct
    enh, spos_row, src2, dest2, g2 = pl.pallas_call(
        _stage1_body,
        out_shape=(sds((S, LLM_H), f32), sds((1, S), i32),
                   sds((NBLK, TBK), i32), sds((NBLK, TBK), i32),
                   sds((NBLK, TBK), f32)),
        interpret=interpret,
    )(triple_embeds, gate_weights[None, :], overlap_cnt[None, :],
      sel_indices[None, :], cu_seqlens[None, :],
      W1, b1[None, :], W2, b2[None, :], ln_g[None, :], ln_b[None, :],
      Wo1, bo1[None, :], Wo2, bo2[None, :])

    out = _build_sc(interpret=interpret)(
        embed_table, token_ids, src2, dest2, g2, enh, spos_row)
    return out[None, :, :]


# speedup vs baseline: 1.0859x; 1.0195x over previous
import jax
import jax.numpy as jnp
from jax import lax
from jax.experimental import pallas as pl
from jax.experimental.pallas import tpu as pltpu
from jax.experimental.pallas import tpu_sc as plsc

S = 256
N_TRIPLES = 4096
TRIPLE_H = 512
LLM_H = 2048
TOTAL_TOK = 8192
GATE_BOOST = 1.0

NC, NS = 2, 16
NW = NC * NS
TOK_W = TOTAL_TOK // NW
K = 16
NCH = TOK_W // K
SROW_W = S // NW
TBK = 512
NBLK = TOTAL_TOK // TBK

f32 = jnp.float32
i32 = jnp.int32


def _stage1_body(te, gw_row, k_row, sel_row, cu_row,
                 W1, b1, W2, b2, ln_g, ln_b, Wo1, bo1, Wo2, bo2,
                 enh_o, spos_o, src_o, dest_o, g_o):
    isub = lax.broadcasted_iota(i32, (S, S), 0).astype(f32)
    ilan = lax.broadcasted_iota(i32, (S, S), 1).astype(f32)
    ident = (isub == ilan).astype(f32)

    def to_col(row):
        return lax.dot_general(ident, row, (((1,), (1,)), ((), ())),
                               preferred_element_type=f32)

    kr = k_row[...].astype(f32)
    kc = to_col(kr)
    cu_f = cu_row[...].astype(f32)
    cu_lo_r, cu_hi_r = cu_f[:, :S], cu_f[:, 1:]
    lens_row = cu_hi_r - cu_lo_r
    lens_col = to_col(cu_hi_r) - to_col(cu_lo_r)

    vsub = lax.broadcasted_iota(i32, (N_TRIPLES, S), 0)
    onehot_selT = (sel_row[...] == vsub).astype(f32)
    selected = lax.dot_general(onehot_selT, te[...], (((0,), (0,)), ((), ())),
                               preferred_element_type=f32)
    g_raw_row = lax.dot_general(gw_row[...], onehot_selT,
                                (((1,), (0,)), ((), ())),
                                preferred_element_type=f32)

    h = jnp.maximum(jnp.dot(selected, W1[...], preferred_element_type=f32)
                    + b1[...], 0.0)
    h = jnp.dot(h, W2[...], preferred_element_type=f32) + b2[...]
    mu = jnp.mean(h, axis=-1, keepdims=True)
    var = jnp.mean((h - mu) ** 2, axis=-1, keepdims=True)
    sp = (h - mu) * lax.rsqrt(var + 1e-5) * ln_g[...] + ln_b[...]

    Mc = (kr < kc) | ((kr == kc) & (ilan < isub))
    rank_col = jnp.sum(Mc.astype(f32), axis=1, keepdims=True)
    Mr = (kc < kr) | ((kc == kr) & (isub < ilan))
    rank_row = jnp.sum(Mr.astype(f32), axis=0, keepdims=True)

    PT = (rank_col == ilan).astype(f32)
    Ppi = (rank_row == isub).astype(f32)

    scnt_col = jnp.sum(Ppi * kr, axis=1, keepdims=True)
    graw_s_col = jnp.sum(Ppi * g_raw_row, axis=1, keepdims=True)
    a_col = jnp.sum(Ppi * cu_lo_r, axis=1, keepdims=True)
    slens_row = jnp.sum(PT * lens_col, axis=0, keepdims=True)
    slens_col = jnp.sum(Ppi * lens_row, axis=1, keepdims=True)

    sorted_struct = lax.dot_general(PT, sp, (((0,), (0,)), ((), ())),
                                    preferred_element_type=f32)

    ncu_row = jnp.sum((isub <= ilan).astype(f32) * slens_col,
                      axis=0, keepdims=True)
    ncu_col = jnp.sum((ilan <= isub).astype(f32) * slens_row,
                      axis=1, keepdims=True)
    start_row = ncu_row - slens_row
    start_col = ncu_col - slens_col
    spos_row = start_row + lax.broadcasted_iota(i32, (1, S), 1).astype(f32)
    spos_o[...] = spos_row.astype(i32)

    maxc = jnp.maximum(jnp.max(kr), 1.0)
    so_col = scnt_col / maxc
    ob = jnp.maximum(so_col * Wo1[...] + bo1[...], 0.0)
    ob = jnp.dot(ob, Wo2[...], preferred_element_type=f32) + bo2[...]
    lowmask_col = scnt_col < 2.0
    ob = jnp.where(lowmask_col, 0.0, ob)
    enh_o[...] = sorted_struct + ob

    g_sel_col = jnp.where(lowmask_col, 1.0, 1.0 + GATE_BOOST * graw_s_col)

    isub_col = lax.broadcasted_iota(i32, (S, 1), 0).astype(f32)
    for blk in range(NBLK):
        t_row = (lax.broadcasted_iota(i32, (1, TBK), 1).astype(f32)
                 + float(blk * TBK))
        ind = (t_row >= ncu_col).astype(f32)
        seg_row = jnp.sum(ind, axis=0, keepdims=True)
        onehot = (seg_row == isub_col).astype(f32)
        start_at = jnp.sum(onehot * start_col, axis=0, keepdims=True)
        a_at = jnp.sum(onehot * a_col, axis=0, keepdims=True)
        g_at = jnp.sum(onehot * g_sel_col, axis=0, keepdims=True)
        row = pl.ds(blk, 1)
        src_o[row, :] = (a_at + t_row - start_at).astype(i32)
        dest_o[row, :] = (t_row + seg_row + 1.0).astype(i32)
        g_o[row, :] = g_at


def _sc_body(embed, tok_ids, src2, dest2, g2, enh, spos2, out,
             tok_v, src_v, g_v, dest_v, ids2_v, dest16_v, rows_v,
             struct_v, spos_full_v, spos8_v, gsem, ssem, hsem):
    wid = lax.axis_index("s") * NC + lax.axis_index("c")
    row = wid // 2
    off = (wid % 2) * TOK_W

    pltpu.sync_copy(tok_ids, tok_v)
    pltpu.sync_copy(src2.at[pl.ds(row, 1), pl.ds(off, TOK_W)], src_v)
    pltpu.sync_copy(g2.at[pl.ds(row, 1), pl.ds(off, TOK_W)], g_v)
    pltpu.sync_copy(dest2.at[pl.ds(row, 1), pl.ds(off, TOK_W)], dest_v)
    pltpu.sync_copy(spos2, spos_full_v)

    for c in range(NCH):
        idx = src_v[0, pl.ds(c * 16, 16)]
        ids2_v[c] = plsc.load_gather(tok_v, [idx])
        dest16_v[c] = dest_v[0, pl.ds(c * 16, 16)]

    struct_g = pltpu.async_copy(enh.at[pl.ds(wid * SROW_W, SROW_W)],
                                struct_v, hsem)
    lane = lax.broadcasted_iota(i32, (16,), 0)
    lo = (wid % 2) * SROW_W
    window = spos_full_v[0, pl.ds((wid // 2) * 16, 16)]
    plsc.store_compressed(spos8_v.at[:], window,
                          mask=(lane >= lo) & (lane < lo + SROW_W))
    spos8 = spos8_v.at[pl.ds(0, SROW_W)]

    NBUF = 3
    zeros16 = jnp.zeros((16,), i32)
    gathers = [None] * NBUF
    scatters = [None] * NBUF

    def issue_gather(c):
        b = c % NBUF
        if scatters[b] is not None:
            scatters[b].wait()
            scatters[b] = None
        gathers[b] = pltpu.async_copy(embed.at[ids2_v.at[c]], rows_v.at[b],
                                      gsem)

    issue_gather(0)
    issue_gather(1)
    struct_g.wait()
    struct_s = pltpu.async_copy(struct_v, out.at[spos8], hsem)
    for c in range(NCH):
        b = c % NBUF
        if c + 2 < NCH:
            issue_gather(c + 2)
        gathers[b].wait()
        gates = [plsc.load_gather(g_v, [zeros16,
                                        jnp.full((16,), c * K + r, i32)])
                 for r in range(K)]

        def mul(j, _, b=b, gates=gates):
            for r in range(K):
                rows_v[b, r, pl.ds(j * 16, 16)] = (
                    rows_v[b, r, pl.ds(j * 16, 16)] * gates[r])
            return 0

        lax.fori_loop(0, LLM_H // 16, mul, 0)
        scatters[b] = pltpu.async_copy(rows_v.at[b], out.at[dest16_v.at[c]],
                                       ssem)

    struct_s.wait()
    for s in scatters:
        if s is not None:
            s.wait()


def _build_sc(interpret=False):
    mesh = plsc.VectorSubcoreMesh(core_axis_name="c", subcore_axis_name="s",
                                  num_cores=NC, num_subcores=NS)
    return pl.kernel(
        _sc_body,
        out_type=jax.ShapeDtypeStruct((S + TOTAL_TOK, LLM_H), f32),
        mesh=mesh,
        scratch_types=[
            pltpu.VMEM((TOTAL_TOK,), i32),
            pltpu.VMEM((1, TOK_W), i32),
            pltpu.VMEM((1, TOK_W), f32),
            pltpu.VMEM((1, TOK_W), i32),
            pltpu.VMEM((NCH, K), i32),
            pltpu.VMEM((NCH, K), i32),
            pltpu.VMEM((3, K, LLM_H), f32),
            pltpu.VMEM((SROW_W, LLM_H), f32),
            pltpu.VMEM((1, S), i32),
            pltpu.VMEM((16,), i32),
            pltpu.SemaphoreType.DMA,
            pltpu.SemaphoreType.DMA,
            pltpu.SemaphoreType.DMA,
        ],
        compiler_params=pltpu.CompilerParams(needs_layout_passes=False),
        interpret=interpret,
    )


def kernel(triple_embeds, overlap_cnt, sel_indices, gate_weights, token_ids,
           cu_seqlens, W1, b1, W2, b2, ln_g, ln_b, Wo1, bo1, Wo2, bo2,
           embed_table, *, interpret=False):
    sds = jax.ShapeDtypeStruct
    enh, spos_row, src2, dest2, g2 = pl.pallas_call(
        _stage1_body,
        out_shape=(sds((S, LLM_H), f32), sds((1, S), i32),
                   sds((NBLK, TBK), i32), sds((NBLK, TBK), i32),
                   sds((NBLK, TBK), f32)),
        interpret=interpret,
    )(triple_embeds, gate_weights[None, :], overlap_cnt[None, :],
      sel_indices[None, :], cu_seqlens[None, :],
      W1, b1[None, :], W2, b2[None, :], ln_g[None, :], ln_b[None, :],
      Wo1, bo1[None, :], Wo2, bo2[None, :])

    out = _build_sc(interpret=interpret)(
        embed_table, token_ids, src2, dest2, g2, enh, spos_row)
    return out[None, :, :]
